# baseline (speedup 1.0000x reference)
import functools

import jax
import jax.numpy as jnp
from jax import lax
from jax.experimental import pallas as pl
from jax.experimental.pallas import tpu as pltpu
from jax.experimental.pallas import tpu_sc as plsc

N_NODES = 10000
N_EDGES = 320000
D_IN = 128
D_H = 64
NV = D_H // 16

NC = 2
NS = 16
NW = NC * NS
BLK = 80
NBLK = 125
EPW = NBLK * BLK
ROWS_PER_TILE = 624
ROWS_TAIL = N_NODES - NS * ROWS_PER_TILE

_mesh = plsc.VectorSubcoreMesh(core_axis_name="c", subcore_axis_name="s")


@functools.partial(
    pl.kernel,
    out_type=jax.ShapeDtypeStruct((NC, N_NODES, D_H), jnp.float32),
    mesh=_mesh,
    compiler_params=pltpu.CompilerParams(use_tc_tiling_on_sc=False),
    scratch_types=[
        pltpu.VMEM((NBLK, BLK), jnp.int32),
        pltpu.VMEM((NBLK, BLK), jnp.int32),
        pltpu.VMEM((NBLK, BLK), jnp.float32),
        pltpu.VMEM((4, BLK, D_H), jnp.float32),
        pltpu.VMEM((4, BLK, D_H), jnp.float32),
        pltpu.VMEM_SHARED((N_NODES, D_H), jnp.float32),
        pltpu.SemaphoreType.DMA,
        pltpu.SemaphoreType.DMA,
        pltpu.SemaphoreType.DMA,
        pltpu.SemaphoreType.DMA,
        pltpu.SemaphoreType.DMA,
        pltpu.SemaphoreType.DMA,
        pltpu.SemaphoreType.DMA,
        pltpu.SemaphoreType.DMA,
    ],
)
def _sc_messages(h_hbm, src_hbm, tgt_hbm, attr_hbm, zeros_hbm, out_hbm,
                 src_v, tgt_v, attr_v, gbuf, sbuf, acc_sh,
                 sem_g0, sem_g1, sem_g2, sem_g3,
                 sem_s0, sem_s1, sem_s2, sem_s3):
    cid = lax.axis_index("c")
    sid = lax.axis_index("s")
    wid = cid * NS + sid
    sem_g = (sem_g0, sem_g1, sem_g2, sem_g3)
    sem_s = (sem_s0, sem_s1, sem_s2, sem_s3)

    r0 = sid * ROWS_PER_TILE
    d_src = pltpu.async_copy(src_hbm.at[wid], src_v, sem_g0)
    d_tgt = pltpu.async_copy(tgt_hbm.at[wid], tgt_v, sem_g1)
    d_att = pltpu.async_copy(attr_hbm.at[wid], attr_v, sem_g2)
    d_zero = pltpu.async_copy(zeros_hbm.at[pl.ds(r0, ROWS_PER_TILE)],
                              acc_sh.at[pl.ds(r0, ROWS_PER_TILE)], sem_g3)

    @pl.when(sid == 0)
    def _zero_tail():
        t0 = NS * ROWS_PER_TILE
        pltpu.sync_copy(zeros_hbm.at[pl.ds(t0, ROWS_TAIL)],
                        acc_sh.at[pl.ds(t0, ROWS_TAIL)])

    d_src.wait()
    d_tgt.wait()
    d_att.wait()
    d_zero.wait()
    plsc.subcore_barrier()

    def start_gather(j, b):
        return pltpu.async_copy(h_hbm.at[src_v.at[j]], gbuf.at[b], sem_g[b])

    def wait_gather(j, b):
        pltpu.make_async_copy(h_hbm.at[src_v.at[j]], gbuf.at[b], sem_g[b]).wait()

    def start_scatter(j, b):
        return pltpu.async_copy(sbuf.at[b], acc_sh.at[tgt_v.at[j]], sem_s[b],
                                add=True)

    def wait_scatter(j, b):
        pltpu.make_async_copy(sbuf.at[b], acc_sh.at[tgt_v.at[j]],
                              sem_s[b]).wait()

    def scale(j, gb, sb):
        @plsc.parallel_loop(0, BLK // 16)
        def grp(g):
            a16 = attr_v[j, pl.ds(g * 16, 16)]
            for i in range(16):
                sv = jnp.full((16,), a16[i], jnp.float32)
                e = g * 16 + i
                for k in range(NV):
                    sbuf[sb, e, pl.ds(k * 16, 16)] = (
                        gbuf[gb, e, pl.ds(k * 16, 16)] * sv)

    descs = [start_gather(j, j) for j in range(4)]
    for j in range(5):
        if j < 4:
            descs[j].wait()
        else:
            wait_gather(j, j % 4)
        if j >= 4:
            wait_scatter(j - 4, j % 4)
        scale(j, j % 4, j % 4)
        start_scatter(j, j % 4)
        start_gather(j + 4, j % 4)

    def body(t, carry):
        for b in range(4):
            j = 5 + 4 * t + b
            gb = (5 + b) % 4
            sb = (5 + b) % 4
            wait_gather(j, gb)
            wait_scatter(j - 4, sb)
            scale(j, gb, sb)
            start_scatter(j, sb)

            @pl.when(j + 4 < NBLK)
            def _():
                start_gather(j + 4, gb)
        return carry

    lax.fori_loop(0, (NBLK - 5) // 4, body, 0)

    for j in range(NBLK - 4, NBLK):
        wait_scatter(j, j % 4)

    plsc.subcore_barrier()
    pltpu.sync_copy(acc_sh.at[pl.ds(r0, ROWS_PER_TILE)],
                    out_hbm.at[cid, pl.ds(r0, ROWS_PER_TILE)])

    @pl.when(sid == 0)
    def _write_tail():
        t0 = NS * ROWS_PER_TILE
        pltpu.sync_copy(acc_sh.at[pl.ds(t0, ROWS_TAIL)],
                        out_hbm.at[cid, pl.ds(t0, ROWS_TAIL)])


def _tc_in_body(x_ref, w_ref, b_ref, o_ref):
    o_ref[...] = jnp.maximum(
        jnp.dot(x_ref[...], w_ref[...], preferred_element_type=jnp.float32)
        + b_ref[...], 0.0)


_tc_in = pl.pallas_call(
    _tc_in_body,
    out_shape=jax.ShapeDtypeStruct((N_NODES, D_H), jnp.float32),
)


def _tc_layer_body(h_ref, m_ref, w_ref, b_ref, o_ref):
    t = h_ref[...] + m_ref[0] + m_ref[1]
    o_ref[...] = jnp.maximum(
        jnp.dot(t, w_ref[...], preferred_element_type=jnp.float32)
        + b_ref[...], 0.0)


_tc_layer = pl.pallas_call(
    _tc_layer_body,
    out_shape=jax.ShapeDtypeStruct((N_NODES, D_H), jnp.float32),
)


def kernel(x, edge_index, edge_attr, W_in, b_in, W1, b1, W2, b2, W3, b3):
    src = edge_index[0].astype(jnp.int32).reshape(NW, NBLK, BLK)
    tgt = edge_index[1].astype(jnp.int32).reshape(NW, NBLK, BLK)
    attr = edge_attr.astype(jnp.float32).reshape(NW, NBLK, BLK)
    zeros = jnp.zeros((N_NODES, D_H), jnp.float32)

    h = _tc_in(x, W_in, b_in.reshape(1, D_H))
    states = [h]
    for (Wl, bl) in ((W1, b1), (W2, b2), (W3, b3)):
        m = _sc_messages(h, src, tgt, attr, zeros)
        h = _tc_layer(h, m, Wl, bl.reshape(1, D_H))
        states.append(h)
    return tuple(states)


# speedup vs baseline: 1.2552x; 1.2552x over previous
import functools

import jax
import jax.numpy as jnp
from jax import lax
from jax.experimental import pallas as pl
from jax.experimental.pallas import tpu as pltpu
from jax.experimental.pallas import tpu_sc as plsc

N_NODES = 10000
N_EDGES = 320000
D_IN = 128
D_H = 64
NV = D_H // 16

NC = 2
NS = 16
NW = NC * NS
BLK = 80
NBLK = 125
EPW = NBLK * BLK
ROWS_PER_TILE = 624
ROWS_TAIL = N_NODES - NS * ROWS_PER_TILE

_mesh = plsc.VectorSubcoreMesh(core_axis_name="c", subcore_axis_name="s")


@functools.partial(
    pl.kernel,
    out_type=jax.ShapeDtypeStruct((NC, N_NODES, D_H), jnp.bfloat16),
    mesh=_mesh,
    compiler_params=pltpu.CompilerParams(use_tc_tiling_on_sc=False,
                                         needs_layout_passes=False),
    scratch_types=[
        pltpu.VMEM((NBLK, BLK), jnp.int32),
        pltpu.VMEM((NBLK, BLK), jnp.int32),
        pltpu.VMEM((NBLK, BLK), jnp.float32),
        pltpu.VMEM((4, BLK, D_H), jnp.bfloat16),
        pltpu.VMEM((4, BLK, D_H), jnp.bfloat16),
        pltpu.VMEM_SHARED((N_NODES, D_H), jnp.bfloat16),
        pltpu.SemaphoreType.DMA,
        pltpu.SemaphoreType.DMA,
        pltpu.SemaphoreType.DMA,
        pltpu.SemaphoreType.DMA,
        pltpu.SemaphoreType.DMA,
        pltpu.SemaphoreType.DMA,
        pltpu.SemaphoreType.DMA,
        pltpu.SemaphoreType.DMA,
    ],
)
def _sc_messages(h_hbm, src_hbm, tgt_hbm, attr_hbm, zeros_hbm, out_hbm,
                 src_v, tgt_v, attr_v, gbuf, sbuf, acc_sh,
                 sem_g0, sem_g1, sem_g2, sem_g3,
                 sem_s0, sem_s1, sem_s2, sem_s3):
    cid = lax.axis_index("c")
    sid = lax.axis_index("s")
    wid = cid * NS + sid
    sem_g = (sem_g0, sem_g1, sem_g2, sem_g3)
    sem_s = (sem_s0, sem_s1, sem_s2, sem_s3)

    r0 = sid * ROWS_PER_TILE
    d_src = pltpu.async_copy(src_hbm.at[wid], src_v, sem_g0)
    d_tgt = pltpu.async_copy(tgt_hbm.at[wid], tgt_v, sem_g1)
    d_att = pltpu.async_copy(attr_hbm.at[wid], attr_v, sem_g2)
    d_zero = pltpu.async_copy(zeros_hbm.at[pl.ds(r0, ROWS_PER_TILE)],
                              acc_sh.at[pl.ds(r0, ROWS_PER_TILE)], sem_g3)

    @pl.when(sid == 0)
    def _zero_tail():
        t0 = NS * ROWS_PER_TILE
        pltpu.sync_copy(zeros_hbm.at[pl.ds(t0, ROWS_TAIL)],
                        acc_sh.at[pl.ds(t0, ROWS_TAIL)])

    d_src.wait()
    d_tgt.wait()
    d_att.wait()
    d_zero.wait()
    plsc.subcore_barrier()

    def start_gather(j, b):
        return pltpu.async_copy(h_hbm.at[src_v.at[j]], gbuf.at[b], sem_g[b])

    def wait_gather(j, b):
        pltpu.make_async_copy(h_hbm.at[src_v.at[j]], gbuf.at[b], sem_g[b]).wait()

    def start_scatter(j, b):
        return pltpu.async_copy(sbuf.at[b], acc_sh.at[tgt_v.at[j]], sem_s[b],
                                add=True)

    def wait_scatter(j, b):
        pltpu.make_async_copy(sbuf.at[b], acc_sh.at[tgt_v.at[j]],
                              sem_s[b]).wait()

    def scale(j, gb, sb):
        def grp(g, c):
            a16 = attr_v[j, pl.ds(g * 16, 16)]
            for i in range(16):
                sv = jnp.full((16,), a16[i], jnp.float32)
                sv2 = plsc.pack(sv, sv, format=plsc.PackFormat.INTERLEAVED)
                e = g * 16 + i
                for k in range(D_H // 32):
                    sbuf[sb, e, pl.ds(k * 32, 32)] = (
                        gbuf[gb, e, pl.ds(k * 32, 32)] * sv2)
            return c
        lax.fori_loop(0, BLK // 16, grp, 0)

    descs = [start_gather(j, j) for j in range(4)]
    for j in range(5):
        if j < 4:
            descs[j].wait()
        else:
            wait_gather(j, j % 4)
        if j >= 4:
            wait_scatter(j - 4, j % 4)
        scale(j, j % 4, j % 4)
        start_scatter(j, j % 4)
        start_gather(j + 4, j % 4)

    def body(t, carry):
        for b in range(4):
            j = 5 + 4 * t + b
            gb = (5 + b) % 4
            sb = (5 + b) % 4
            wait_gather(j, gb)
            wait_scatter(j - 4, sb)
            scale(j, gb, sb)
            start_scatter(j, sb)

            @pl.when(j + 4 < NBLK)
            def _():
                start_gather(j + 4, gb)
        return carry

    lax.fori_loop(0, (NBLK - 5) // 4, body, 0)

    for j in range(NBLK - 4, NBLK):
        wait_scatter(j, j % 4)

    plsc.subcore_barrier()
    pltpu.sync_copy(acc_sh.at[pl.ds(r0, ROWS_PER_TILE)],
                    out_hbm.at[cid, pl.ds(r0, ROWS_PER_TILE)])

    @pl.when(sid == 0)
    def _write_tail():
        t0 = NS * ROWS_PER_TILE
        pltpu.sync_copy(acc_sh.at[pl.ds(t0, ROWS_TAIL)],
                        out_hbm.at[cid, pl.ds(t0, ROWS_TAIL)])


def _tc_in_body(x_ref, w_ref, b_ref, o_ref, ob_ref):
    h = jnp.maximum(
        jnp.dot(x_ref[...], w_ref[...], preferred_element_type=jnp.float32)
        + b_ref[...], 0.0)
    o_ref[...] = h
    ob_ref[...] = h.astype(jnp.bfloat16)


_tc_in = pl.pallas_call(
    _tc_in_body,
    out_shape=(jax.ShapeDtypeStruct((N_NODES, D_H), jnp.float32),
               jax.ShapeDtypeStruct((N_NODES, D_H), jnp.bfloat16)),
)


def _tc_layer_body(h_ref, m_ref, w_ref, b_ref, o_ref, ob_ref):
    t = (h_ref[...] + m_ref[0].astype(jnp.float32)
         + m_ref[1].astype(jnp.float32))
    h = jnp.maximum(
        jnp.dot(t, w_ref[...], preferred_element_type=jnp.float32)
        + b_ref[...], 0.0)
    o_ref[...] = h
    ob_ref[...] = h.astype(jnp.bfloat16)


_tc_layer = pl.pallas_call(
    _tc_layer_body,
    out_shape=(jax.ShapeDtypeStruct((N_NODES, D_H), jnp.float32),
               jax.ShapeDtypeStruct((N_NODES, D_H), jnp.bfloat16)),
)


def kernel(x, edge_index, edge_attr, W_in, b_in, W1, b1, W2, b2, W3, b3):
    src = edge_index[0].astype(jnp.int32).reshape(NW, NBLK, BLK)
    tgt = edge_index[1].astype(jnp.int32).reshape(NW, NBLK, BLK)
    attr = edge_attr.astype(jnp.float32).reshape(NW, NBLK, BLK)
    zeros = jnp.zeros((N_NODES, D_H), jnp.bfloat16)

    h, hb = _tc_in(x, W_in, b_in.reshape(1, D_H))
    states = [h]
    for (Wl, bl) in ((W1, b1), (W2, b2), (W3, b3)):
        m = _sc_messages(hb, src, tgt, attr, zeros)
        h, hb = _tc_layer(h, m, Wl, bl.reshape(1, D_H))
        states.append(h)
    return tuple(states)


# speedup vs baseline: 1.3164x; 1.0488x over previous
import functools

import jax
import jax.numpy as jnp
from jax import lax
from jax.experimental import pallas as pl
from jax.experimental.pallas import tpu as pltpu
from jax.experimental.pallas import tpu_sc as plsc

N_NODES = 10000
N_EDGES = 320000
D_IN = 128
D_H = 64
NV = D_H // 16

NC = 2
NS = 16
NW = NC * NS
BLK = 80
NBLK = 125
EPW = NBLK * BLK
ROWS_PER_TILE = 624
ROWS_TAIL = N_NODES - NS * ROWS_PER_TILE

_mesh = plsc.VectorSubcoreMesh(core_axis_name="c", subcore_axis_name="s")


@functools.partial(
    pl.kernel,
    out_type=jax.ShapeDtypeStruct((NC, N_NODES, D_H), jnp.bfloat16),
    mesh=_mesh,
    compiler_params=pltpu.CompilerParams(use_tc_tiling_on_sc=False,
                                         needs_layout_passes=False),
    scratch_types=[
        pltpu.VMEM((NBLK, BLK), jnp.int32),
        pltpu.VMEM((NBLK, BLK), jnp.int32),
        pltpu.VMEM((NBLK, BLK), jnp.float32),
        pltpu.VMEM((8, BLK, D_H), jnp.bfloat16),
        pltpu.VMEM((4, BLK, D_H), jnp.bfloat16),
        pltpu.VMEM_SHARED((N_NODES, D_H), jnp.bfloat16),
        pltpu.SemaphoreType.DMA,
        pltpu.SemaphoreType.DMA,
        pltpu.SemaphoreType.DMA,
        pltpu.SemaphoreType.DMA,
        pltpu.SemaphoreType.DMA,
        pltpu.SemaphoreType.DMA,
        pltpu.SemaphoreType.DMA,
        pltpu.SemaphoreType.DMA,
        pltpu.SemaphoreType.DMA,
        pltpu.SemaphoreType.DMA,
        pltpu.SemaphoreType.DMA,
        pltpu.SemaphoreType.DMA,
    ],
)
def _sc_messages(h_hbm, src_hbm, tgt_hbm, attr_hbm, zeros_hbm, out_hbm,
                 src_v, tgt_v, attr_v, gbuf, sbuf, acc_sh,
                 sem_g0, sem_g1, sem_g2, sem_g3,
                 sem_g4, sem_g5, sem_g6, sem_g7,
                 sem_s0, sem_s1, sem_s2, sem_s3):
    cid = lax.axis_index("c")
    sid = lax.axis_index("s")
    wid = cid * NS + sid
    sem_g = (sem_g0, sem_g1, sem_g2, sem_g3, sem_g4, sem_g5, sem_g6, sem_g7)
    sem_s = (sem_s0, sem_s1, sem_s2, sem_s3)

    r0 = sid * ROWS_PER_TILE
    d_src = pltpu.async_copy(src_hbm.at[wid], src_v, sem_g0)
    d_tgt = pltpu.async_copy(tgt_hbm.at[wid], tgt_v, sem_g1)
    d_att = pltpu.async_copy(attr_hbm.at[wid], attr_v, sem_g2)
    d_zero = pltpu.async_copy(zeros_hbm.at[pl.ds(r0, ROWS_PER_TILE)],
                              acc_sh.at[pl.ds(r0, ROWS_PER_TILE)], sem_g3)

    @pl.when(sid == 0)
    def _zero_tail():
        t0 = NS * ROWS_PER_TILE
        pltpu.sync_copy(zeros_hbm.at[pl.ds(t0, ROWS_TAIL)],
                        acc_sh.at[pl.ds(t0, ROWS_TAIL)])

    d_src.wait()
    d_tgt.wait()
    d_att.wait()
    d_zero.wait()
    plsc.subcore_barrier()

    def start_gather(j, b):
        return pltpu.async_copy(h_hbm.at[src_v.at[j]], gbuf.at[b], sem_g[b])

    def wait_gather(j, b):
        pltpu.make_async_copy(h_hbm.at[src_v.at[j]], gbuf.at[b], sem_g[b]).wait()

    def start_scatter(j, b):
        return pltpu.async_copy(sbuf.at[b], acc_sh.at[tgt_v.at[j]], sem_s[b],
                                add=True)

    def wait_scatter(j, b):
        pltpu.make_async_copy(sbuf.at[b], acc_sh.at[tgt_v.at[j]],
                              sem_s[b]).wait()

    def scale(j, gb, sb):
        def grp(g, c):
            a16 = attr_v[j, pl.ds(g * 16, 16)]
            for i in range(16):
                sv = jnp.full((16,), a16[i], jnp.float32)
                sv2 = plsc.pack(sv, sv, format=plsc.PackFormat.INTERLEAVED)
                e = g * 16 + i
                for k in range(D_H // 32):
                    sbuf[sb, e, pl.ds(k * 32, 32)] = (
                        gbuf[gb, e, pl.ds(k * 32, 32)] * sv2)
            return c
        lax.fori_loop(0, BLK // 16, grp, 0)

    descs = [start_gather(j, j) for j in range(8)]
    for j in range(5):
        descs[j].wait()
        if j >= 4:
            wait_scatter(j - 4, j % 4)
        scale(j, j, j % 4)
        start_scatter(j, j % 4)
        start_gather(j + 8, j)

    def body(t, carry):
        for b in range(8):
            j = 5 + 8 * t + b
            gb = (5 + b) % 8
            sb = (5 + b) % 4
            wait_gather(j, gb)
            wait_scatter(j - 4, sb)
            scale(j, gb, sb)
            start_scatter(j, sb)

            @pl.when(j + 8 < NBLK)
            def _():
                start_gather(j + 8, gb)
        return carry

    lax.fori_loop(0, (NBLK - 5) // 8, body, 0)

    for j in range(NBLK - 4, NBLK):
        wait_scatter(j, j % 4)

    plsc.subcore_barrier()
    pltpu.sync_copy(acc_sh.at[pl.ds(r0, ROWS_PER_TILE)],
                    out_hbm.at[cid, pl.ds(r0, ROWS_PER_TILE)])

    @pl.when(sid == 0)
    def _write_tail():
        t0 = NS * ROWS_PER_TILE
        pltpu.sync_copy(acc_sh.at[pl.ds(t0, ROWS_TAIL)],
                        out_hbm.at[cid, pl.ds(t0, ROWS_TAIL)])


def _tc_in_body(x_ref, w_ref, b_ref, o_ref, ob_ref):
    h = jnp.maximum(
        jnp.dot(x_ref[...], w_ref[...], preferred_element_type=jnp.float32)
        + b_ref[...], 0.0)
    o_ref[...] = h
    ob_ref[...] = h.astype(jnp.bfloat16)


_tc_in = pl.pallas_call(
    _tc_in_body,
    out_shape=(jax.ShapeDtypeStruct((N_NODES, D_H), jnp.float32),
               jax.ShapeDtypeStruct((N_NODES, D_H), jnp.bfloat16)),
)


def _tc_layer_body(h_ref, m_ref, w_ref, b_ref, o_ref, ob_ref):
    t = (h_ref[...] + m_ref[0].astype(jnp.float32)
         + m_ref[1].astype(jnp.float32))
    h = jnp.maximum(
        jnp.dot(t, w_ref[...], preferred_element_type=jnp.float32)
        + b_ref[...], 0.0)
    o_ref[...] = h
    ob_ref[...] = h.astype(jnp.bfloat16)


_tc_layer = pl.pallas_call(
    _tc_layer_body,
    out_shape=(jax.ShapeDtypeStruct((N_NODES, D_H), jnp.float32),
               jax.ShapeDtypeStruct((N_NODES, D_H), jnp.bfloat16)),
)


def kernel(x, edge_index, edge_attr, W_in, b_in, W1, b1, W2, b2, W3, b3):
    src = edge_index[0].astype(jnp.int32).reshape(NW, NBLK, BLK)
    tgt = edge_index[1].astype(jnp.int32).reshape(NW, NBLK, BLK)
    attr = edge_attr.astype(jnp.float32).reshape(NW, NBLK, BLK)
    zeros = jnp.zeros((N_NODES, D_H), jnp.bfloat16)

    h, hb = _tc_in(x, W_in, b_in.reshape(1, D_H))
    states = [h]
    for (Wl, bl) in ((W1, b1), (W2, b2), (W3, b3)):
        m = _sc_messages(hb, src, tgt, attr, zeros)
        h, hb = _tc_layer(h, m, Wl, bl.reshape(1, D_H))
        states.append(h)
    return tuple(states)
